# Initial kernel scaffold; baseline (speedup 1.0000x reference)
import jax
import jax.numpy as jnp
from jax.experimental import pallas as pl


def kernel(move_ids, table):
    raise NotImplementedError("write your pallas kernel here")


# speedup vs baseline: 4.0908x; 4.0908x over previous
import functools

import jax
import jax.numpy as jnp
from jax import lax
from jax.experimental import pallas as pl
from jax.experimental.pallas import tpu as pltpu
from jax.experimental.pallas import tpu_sc as plsc

EMBED_D = 64
CHUNK = 128


@functools.cache
def _make(b_flat, nc, ns):
    nw = nc * ns
    b_per_w = b_flat // nw
    n_chunks = b_per_w // CHUNK
    mesh = plsc.VectorSubcoreMesh(core_axis_name="c", subcore_axis_name="s")

    @functools.partial(
        pl.kernel,
        mesh=mesh,
        out_type=jax.ShapeDtypeStruct((b_flat, EMBED_D), jnp.float32),
        scratch_types=[
            pltpu.VMEM((n_chunks, CHUNK), jnp.int32),
            pltpu.VMEM((CHUNK, EMBED_D), jnp.float32),
            pltpu.SemaphoreType.DMA,
        ],
        compiler_params=pltpu.CompilerParams(use_tc_tiling_on_sc=False),
    )
    def k(table_hbm, idx_hbm, out_hbm, idx_v, rows_v, sem):
        wid = lax.axis_index("s") * nc + lax.axis_index("c")
        pltpu.sync_copy(idx_hbm.at[wid], idx_v)
        base = wid * b_per_w

        def body(j, carry):
            pltpu.async_copy(table_hbm.at[idx_v.at[j]], rows_v, sem).wait()
            pltpu.sync_copy(rows_v, out_hbm.at[pl.ds(base + j * CHUNK, CHUNK)])
            return carry

        lax.fori_loop(0, n_chunks, body, 0)

    return k


def kernel(move_ids, table):
    b, h = move_ids.shape
    info = plsc.get_sparse_core_info()
    nc, ns = info.num_cores, info.num_subcores
    nw = nc * ns
    b_flat = b * h
    idx3 = move_ids.astype(jnp.int32).reshape(nw, b_flat // nw // CHUNK, CHUNK)
    out = _make(b_flat, nc, ns)(table, idx3)
    return out.reshape(b, h, EMBED_D)


# speedup vs baseline: 4.6765x; 1.1432x over previous
import functools

import jax
import jax.numpy as jnp
from jax import lax
from jax.experimental import pallas as pl
from jax.experimental.pallas import tpu as pltpu
from jax.experimental.pallas import tpu_sc as plsc

EMBED_D = 64
CHUNK = 128
NBUF = 5
PF = 3


@functools.cache
def _make(b_flat, nc, ns):
    nw = nc * ns
    b_per_w = b_flat // nw
    n_chunks = b_per_w // CHUNK
    assert n_chunks % NBUF == 0 and n_chunks >= 2 * NBUF
    mesh = plsc.VectorSubcoreMesh(core_axis_name="c", subcore_axis_name="s")

    @functools.partial(
        pl.kernel,
        mesh=mesh,
        out_type=jax.ShapeDtypeStruct((b_flat, EMBED_D), jnp.float32),
        scratch_types=[
            pltpu.VMEM((n_chunks, CHUNK), jnp.int32),
            pltpu.VMEM((NBUF, CHUNK, EMBED_D), jnp.float32),
            [pltpu.SemaphoreType.DMA] * NBUF,
            [pltpu.SemaphoreType.DMA] * NBUF,
        ],
        compiler_params=pltpu.CompilerParams(use_tc_tiling_on_sc=False),
    )
    def k(table_hbm, idx_hbm, out_hbm, idx_v, rows_v, sem_g, sem_w):
        wid = lax.axis_index("s") * nc + lax.axis_index("c")
        pltpu.sync_copy(idx_hbm.at[wid], idx_v)
        base = wid * b_per_w

        def gather(j, b):
            pltpu.make_async_copy(
                table_hbm.at[idx_v.at[j]], rows_v.at[b], sem_g[b]
            ).start()

        def gather_wait(b):
            pltpu.make_async_copy(
                table_hbm.at[idx_v.at[0]], rows_v.at[b], sem_g[b]
            ).wait()

        def writeback(j, b):
            pltpu.make_async_copy(
                rows_v.at[b], out_hbm.at[pl.ds(base + j * CHUNK, CHUNK)], sem_w[b]
            ).start()

        def writeback_wait(b):
            pltpu.make_async_copy(
                rows_v.at[b], out_hbm.at[pl.ds(base, CHUNK)], sem_w[b]
            ).wait()

        for b in range(PF):
            gather(b, b)

        def body(g, carry):
            for b in range(NBUF):
                j = g * NBUF + b
                gather_wait(b)
                writeback(j, b)
                nj = j + PF
                bb = (b + PF) % NBUF

                @pl.when(nj < n_chunks)
                def _():
                    @pl.when(nj >= NBUF)
                    def _():
                        writeback_wait(bb)

                    gather(nj, bb)

            return carry

        lax.fori_loop(0, n_chunks // NBUF, body, 0)

        for b in range(NBUF):
            writeback_wait(b)

    return k


def kernel(move_ids, table):
    b, h = move_ids.shape
    info = plsc.get_sparse_core_info()
    nc, ns = info.num_cores, info.num_subcores
    nw = nc * ns
    b_flat = b * h
    idx3 = move_ids.astype(jnp.int32).reshape(nw, b_flat // nw // CHUNK, CHUNK)
    out = _make(b_flat, nc, ns)(table, idx3)
    return out.reshape(b, h, EMBED_D)


# speedup vs baseline: 6.8983x; 1.4751x over previous
import functools

import jax
import jax.numpy as jnp
from jax import lax
from jax.experimental import pallas as pl
from jax.experimental.pallas import tpu as pltpu
from jax.experimental.pallas import tpu_sc as plsc

EMBED_D = 64
PAD_D = 128
PAD_H = 56
NBUF = 8
PF = 4


@functools.cache
def _make(batch, hist, nc, ns):
    nw = nc * ns
    b_per_w = batch // nw
    assert b_per_w % NBUF == 0 and b_per_w >= 2 * NBUF
    mesh = plsc.VectorSubcoreMesh(core_axis_name="c", subcore_axis_name="s")

    @functools.partial(
        pl.kernel,
        mesh=mesh,
        out_type=jax.ShapeDtypeStruct((batch, PAD_H, PAD_D), jnp.float32),
        scratch_types=[
            pltpu.VMEM((b_per_w, hist), jnp.int32),
            pltpu.VMEM((NBUF, hist, EMBED_D), jnp.float32),
            [pltpu.SemaphoreType.DMA] * NBUF,
            [pltpu.SemaphoreType.DMA] * NBUF,
        ],
        compiler_params=pltpu.CompilerParams(use_tc_tiling_on_sc=False),
    )
    def k(table_hbm, idx_hbm, out_hbm, idx_v, rows_v, sem_g, sem_w):
        wid = lax.axis_index("s") * nc + lax.axis_index("c")
        base = wid * b_per_w
        pltpu.sync_copy(idx_hbm.at[pl.ds(base, b_per_w)], idx_v)

        def gather(t, b):
            pltpu.make_async_copy(
                table_hbm.at[idx_v.at[t]], rows_v.at[b], sem_g[b]
            ).start()

        def gather_wait(b):
            pltpu.make_async_copy(
                table_hbm.at[idx_v.at[0]], rows_v.at[b], sem_g[b]
            ).wait()

        def writeback(t, b):
            pltpu.make_async_copy(
                rows_v.at[b],
                out_hbm.at[base + t, pl.ds(0, hist), pl.ds(0, EMBED_D)],
                sem_w[b],
            ).start()

        def writeback_wait(b):
            pltpu.make_async_copy(
                rows_v.at[b],
                out_hbm.at[base, pl.ds(0, hist), pl.ds(0, EMBED_D)],
                sem_w[b],
            ).wait()

        for b in range(PF):
            gather(b, b)

        def body(g, carry):
            for b in range(NBUF):
                t = g * NBUF + b
                gather_wait(b)
                writeback(t, b)
                nt = t + PF
                bb = (b + PF) % NBUF

                @pl.when(nt < b_per_w)
                def _():
                    @pl.when(nt >= NBUF)
                    def _():
                        writeback_wait(bb)

                    gather(nt, bb)

            return carry

        lax.fori_loop(0, b_per_w // NBUF, body, 0)

        for b in range(NBUF):
            writeback_wait(b)

    return k


def kernel(move_ids, table):
    batch, hist = move_ids.shape
    info = plsc.get_sparse_core_info()
    nc, ns = info.num_cores, info.num_subcores
    out3 = _make(batch, hist, nc, ns)(table, move_ids.astype(jnp.int32))
    return lax.slice(out3, (0, 0, 0), (batch, hist, EMBED_D))
